# Initial kernel scaffold; baseline (speedup 1.0000x reference)
import jax
import jax.numpy as jnp
from jax.experimental import pallas as pl


def kernel(xr, x):
    raise NotImplementedError("write your pallas kernel here")


# speedup vs baseline: 1.0331x; 1.0331x over previous
import functools

import jax
import jax.numpy as jnp
from jax import lax
from jax.experimental import pallas as pl
from jax.experimental.pallas import tpu as pltpu


def _mse_body(xr_ref, x_ref, out_ref):
    c = pl.program_id(1)

    @pl.when(c == 0)
    def _():
        out_ref[...] = jnp.zeros_like(out_ref)

    xb = x_ref[0]
    d0 = xr_ref[0, 0] - xb
    d1 = xr_ref[1, 0] - xb
    s0 = jnp.sum(d0 * d0)
    s1 = jnp.sum(d1 * d1)
    lane = lax.broadcasted_iota(jnp.int32, (1, 1, 2), 2)
    out_ref[...] += jnp.where(lane == 0, s0, s1)


def _combine_body(l_ref, out_ref, *, inv):
    arr = l_ref[:, 0, :]
    b = arr.shape[0]
    idx = lax.broadcasted_iota(jnp.int32, (b, 2), 0)
    col = lax.broadcasted_iota(jnp.int32, (b, 2), 1)
    neg = float("-inf")
    m0 = jnp.max(jnp.where(col == 0, arr, neg))
    m1 = jnp.max(jnp.where(col == 1, arr, neg))
    j0 = jnp.max(jnp.where((col == 0) & (arr == m0), idx, -1))
    j1 = jnp.max(jnp.where((col == 1) & (arr == m1), idx, -1))
    s0 = jnp.sum(jnp.where(col == 0, arr, 0.0))
    s1 = jnp.sum(jnp.where(col == 1, arr, 0.0))
    d0 = jnp.sum(jnp.where((col == 0) & (idx == j1), arr, 0.0))
    d1 = jnp.sum(jnp.where((col == 1) & (idx == j0), arr, 0.0))
    loss = (s0 - d0 + s1 - d1) * inv
    out_ref[...] = jnp.full((1, 1), loss, jnp.float32)


def _pick_chunk(c0, h, w, budget_bytes=3 * 1024 * 1024):
    best = 1
    for g in range(1, c0 + 1):
        if c0 % g == 0 and g * h * w * 4 <= budget_bytes:
            best = g
    return best


def kernel(xr, x):
    B, C0, H, W = x.shape
    N = C0 * H * W
    G = _pick_chunk(C0, H, W)
    C = C0 // G

    sums = pl.pallas_call(
        _mse_body,
        grid=(B, C),
        in_specs=[
            pl.BlockSpec((2, 1, G, H, W), lambda b, c: (0, b, c, 0, 0)),
            pl.BlockSpec((1, G, H, W), lambda b, c: (b, c, 0, 0)),
        ],
        out_specs=pl.BlockSpec((1, 1, 2), lambda b, c: (b, 0, 0)),
        out_shape=jax.ShapeDtypeStruct((B, 1, 2), jnp.float32),
    )(xr, x)

    rem = int(B * 0.9)
    inv = 1.0 / (rem * N)
    loss = pl.pallas_call(
        functools.partial(_combine_body, inv=inv),
        out_shape=jax.ShapeDtypeStruct((1, 1), jnp.float32),
    )(sums)
    return loss[0, 0]


# speedup vs baseline: 1.0439x; 1.0105x over previous
import jax
import jax.numpy as jnp
from jax import lax
from jax.experimental import pallas as pl
from jax.experimental.pallas import tpu as pltpu


def _body(xr_ref, x_ref, out_ref, acc_ref, *, nb, nc, inv):
    b = pl.program_id(0)
    c = pl.program_id(1)

    @pl.when((b == 0) & (c == 0))
    def _():
        acc_ref[...] = jnp.zeros_like(acc_ref)

    xb = x_ref[0]
    d0 = xr_ref[0, 0] - xb
    d1 = xr_ref[1, 0] - xb
    s0 = jnp.sum(d0 * d0)
    s1 = jnp.sum(d1 * d1)
    lane = lax.broadcasted_iota(jnp.int32, (1, 2), 1)
    acc_ref[pl.ds(b, 1), :] += jnp.where(lane == 0, s0, s1)

    @pl.when((b == nb - 1) & (c == nc - 1))
    def _():
        arr = acc_ref[...]
        idx = lax.broadcasted_iota(jnp.int32, arr.shape, 0)
        col = lax.broadcasted_iota(jnp.int32, arr.shape, 1)
        neg = float("-inf")
        m0 = jnp.max(jnp.where(col == 0, arr, neg))
        m1 = jnp.max(jnp.where(col == 1, arr, neg))
        j0 = jnp.max(jnp.where((col == 0) & (arr == m0), idx, -1))
        j1 = jnp.max(jnp.where((col == 1) & (arr == m1), idx, -1))
        s0t = jnp.sum(jnp.where(col == 0, arr, 0.0))
        s1t = jnp.sum(jnp.where(col == 1, arr, 0.0))
        d0t = jnp.sum(jnp.where((col == 0) & (idx == j1), arr, 0.0))
        d1t = jnp.sum(jnp.where((col == 1) & (idx == j0), arr, 0.0))
        loss = (s0t - d0t + s1t - d1t) * inv
        out_ref[...] = jnp.full((1, 1), loss, jnp.float32)


def _pick_chunk(c0, h, w, budget_bytes=3 * 1024 * 1024):
    best = 1
    for g in range(1, c0 + 1):
        if c0 % g == 0 and g * h * w * 4 <= budget_bytes:
            best = g
    return best


def kernel(xr, x):
    import functools

    B, C0, H, W = x.shape
    N = C0 * H * W
    G = _pick_chunk(C0, H, W)
    C = C0 // G
    rem = int(B * 0.9)
    inv = 1.0 / (rem * N)

    loss = pl.pallas_call(
        functools.partial(_body, nb=B, nc=C, inv=inv),
        grid=(B, C),
        in_specs=[
            pl.BlockSpec((2, 1, G, H, W), lambda b, c: (0, b, c, 0, 0)),
            pl.BlockSpec((1, G, H, W), lambda b, c: (b, c, 0, 0)),
        ],
        out_specs=pl.BlockSpec((1, 1), lambda b, c: (0, 0)),
        out_shape=jax.ShapeDtypeStruct((1, 1), jnp.float32),
        scratch_shapes=[pltpu.VMEM((B, 2), jnp.float32)],
    )(xr, x)
    return loss[0, 0]
